# baseline (speedup 1.0000x reference)
import functools

import jax
import jax.numpy as jnp
from jax import lax
from jax.experimental import pallas as pl
from jax.experimental.pallas import tpu as pltpu
from jax.experimental.pallas import tpu_sc as plsc

BATCH = 512
D = 64
NC = 2
NS = 16
NW = NC * NS


def _sc_segment_sums(text, emb_table, c):
  sw = BATCH // NW
  tw = sw * c
  mesh = plsc.VectorSubcoreMesh(core_axis_name="c", subcore_axis_name="s")

  @functools.partial(
      pl.kernel,
      mesh=mesh,
      compiler_params=pltpu.CompilerParams(use_tc_tiling_on_sc=False),
      out_type=jax.ShapeDtypeStruct((BATCH, D), jnp.float32),
      scratch_types=[
          pltpu.VMEM((tw,), jnp.int32),
          pltpu.VMEM((2, c, D), jnp.float32),
          pltpu.VMEM((sw, D), jnp.float32),
          pltpu.SemaphoreType.DMA,
          pltpu.SemaphoreType.DMA,
      ],
  )
  def k(text_hbm, table_hbm, out_hbm, idx_v, rows_v, acc_v, sem0, sem1):
    wid = lax.axis_index("s") * NC + lax.axis_index("c")
    tbase = wid * tw
    pltpu.sync_copy(text_hbm.at[pl.ds(tbase, tw)], idx_v)
    sems = (sem0, sem1)

    def start(s, buf):
      pltpu.async_copy(
          table_hbm.at[idx_v.at[pl.ds(s * c, c)]], rows_v.at[buf], sems[buf])

    def wait(buf):
      pltpu.make_async_copy(
          table_hbm.at[idx_v.at[pl.ds(0, c)]], rows_v.at[buf],
          sems[buf]).wait()

    start(0, 0)

    def seg_body(s2, carry):
      for b in range(2):
        s = s2 * 2 + b

        @pl.when(s + 1 < sw)
        def _():
          start(s + 1, 1 - b)

        wait(b)

        def acc_body(r, accs):
          return tuple(
              accs[d] + rows_v[b, r, pl.ds(d * 16, 16)] for d in range(4))

        accs = lax.fori_loop(
            0, c, acc_body,
            tuple(jnp.zeros((16,), jnp.float32) for _ in range(4)),
            unroll=8)
        for d in range(4):
          acc_v[s, pl.ds(d * 16, 16)] = accs[d]
      return carry

    lax.fori_loop(0, sw // 2, seg_body, 0)
    pltpu.sync_copy(acc_v, out_hbm.at[pl.ds(wid * sw, sw)])

  return k(text, emb_table)


def _tc_project(sums, wt, b2):
  def body(s_ref, w_ref, b_ref, o_ref):
    o_ref[...] = jnp.dot(
        s_ref[...], w_ref[...], preferred_element_type=jnp.float32) + b_ref[...]

  return pl.pallas_call(
      body,
      out_shape=jax.ShapeDtypeStruct((BATCH, b2.shape[-1]), jnp.float32),
  )(sums, wt, b2)


def kernel(text, emb_table, fc_w, fc_b):
  n = text.shape[0]
  c = n // BATCH
  assert BATCH * c == n and c % 8 == 0 and emb_table.shape[1] == D
  sums = _sc_segment_sums(text.astype(jnp.int32), emb_table, c)
  wt = fc_w.T.astype(jnp.float32) / jnp.float32(c)
  return _tc_project(sums, wt, fc_b.reshape(1, -1).astype(jnp.float32))
